# baseline (speedup 1.0000x reference)
import functools

import jax
import jax.numpy as jnp
from jax import lax
from jax.experimental import pallas as pl
from jax.experimental.pallas import tpu as pltpu
from jax.experimental.pallas import tpu_sc as plsc

N_FIELDS = 26
VOCAB = 100000
N_EMBD = 32
BATCH = 4096

NC = 2
NS = 16
NW = NC * NS
BPW = BATCH // NW
LANES = 16


def _body(idx_hbm, table_hbm, out_hbm, idx_v, rows_v, out_v, sem):
    cid = lax.axis_index("c")
    sid = lax.axis_index("s")
    wid = sid * NC + cid
    base = wid * BPW

    pltpu.sync_copy(idx_hbm.at[:, pl.ds(base, BPW)], idx_v)

    copies = []
    for f in range(N_FIELDS):
        cp = pltpu.make_async_copy(table_hbm.at[idx_v.at[f]], rows_v.at[f], sem)
        cp.start()
        copies.append(cp)
    for cp in copies:
        cp.wait()

    def body_j(j, carry):
        for d in (0, LANES):
            acc = rows_v[0, j, pl.ds(d, LANES)]
            for f in range(1, N_FIELDS):
                acc = acc + rows_v[f, j, pl.ds(d, LANES)]
            out_v[j, pl.ds(d, LANES)] = acc
        return carry

    lax.fori_loop(0, BPW, body_j, 0)

    pltpu.sync_copy(out_v, out_hbm.at[pl.ds(base, BPW)])


@jax.jit
def kernel(latents, tables):
    idx = latents.astype(jnp.int32).T + (
        jnp.arange(N_FIELDS, dtype=jnp.int32) * VOCAB
    )[:, None]
    flat_tables = tables.reshape(N_FIELDS * VOCAB, N_EMBD)

    mesh = plsc.VectorSubcoreMesh(
        core_axis_name="c", subcore_axis_name="s", num_cores=NC, num_subcores=NS
    )
    run = pl.kernel(
        _body,
        out_type=jax.ShapeDtypeStruct((BATCH, N_EMBD), jnp.float32),
        mesh=mesh,
        scratch_types=[
            pltpu.VMEM((N_FIELDS, BPW), jnp.int32),
            pltpu.VMEM((N_FIELDS, BPW, N_EMBD), jnp.float32),
            pltpu.VMEM((BPW, N_EMBD), jnp.float32),
            pltpu.SemaphoreType.DMA,
        ],
        compiler_params=pltpu.CompilerParams(use_tc_tiling_on_sc=False),
    )
    return run(idx, flat_tables)


# speedup vs baseline: 1.0016x; 1.0016x over previous
import functools

import jax
import jax.numpy as jnp
from jax import lax
from jax.experimental import pallas as pl
from jax.experimental.pallas import tpu as pltpu
from jax.experimental.pallas import tpu_sc as plsc

N_FIELDS = 26
VOCAB = 100000
N_EMBD = 32
BATCH = 4096

NC = 2
NS = 16
NW = NC * NS
BPW = BATCH // NW
LANES = 16


def _body(idx_hbm, table_hbm, out_hbm, idx_v, rows_v, out_v, sem):
    cid = lax.axis_index("c")
    sid = lax.axis_index("s")
    wid = sid * NC + cid
    base = wid * BPW

    pltpu.sync_copy(idx_hbm.at[:, pl.ds(base, BPW)], idx_v)

    copies = []
    for f in range(N_FIELDS):
        cp = pltpu.make_async_copy(
            table_hbm.at[f].at[idx_v.at[f]], rows_v.at[f], sem
        )
        cp.start()
        copies.append(cp)
    for cp in copies:
        cp.wait()

    def body_j(j, carry):
        for d in (0, LANES):
            acc = rows_v[0, j, pl.ds(d, LANES)]
            for f in range(1, N_FIELDS):
                acc = acc + rows_v[f, j, pl.ds(d, LANES)]
            out_v[j, pl.ds(d, LANES)] = acc
        return carry

    lax.fori_loop(0, BPW, body_j, 0)

    pltpu.sync_copy(out_v, out_hbm.at[pl.ds(base, BPW)])


@jax.jit
def kernel(latents, tables):
    idx = latents.astype(jnp.int32).T

    mesh = plsc.VectorSubcoreMesh(
        core_axis_name="c", subcore_axis_name="s", num_cores=NC, num_subcores=NS
    )
    run = pl.kernel(
        _body,
        out_type=jax.ShapeDtypeStruct((BATCH, N_EMBD), jnp.float32),
        mesh=mesh,
        scratch_types=[
            pltpu.VMEM((N_FIELDS, BPW), jnp.int32),
            pltpu.VMEM((N_FIELDS, BPW, N_EMBD), jnp.float32),
            pltpu.VMEM((BPW, N_EMBD), jnp.float32),
            pltpu.SemaphoreType.DMA,
        ],
        compiler_params=pltpu.CompilerParams(use_tc_tiling_on_sc=False),
    )
    return run(idx, tables)
